# Initial kernel scaffold; baseline (speedup 1.0000x reference)
import jax
import jax.numpy as jnp
from jax.experimental import pallas as pl


def kernel(frag_feature, attn_bias, spatial_pos, edge_input, attn_edge_type, edge_enc_w, edge_dis_w, spatial_enc_w, vdist_w):
    raise NotImplementedError("write your pallas kernel here")


# speedup vs baseline: 25.0180x; 25.0180x over previous
import functools

import jax
import jax.numpy as jnp
import numpy as np
from jax import lax
from jax.experimental import pallas as pl
from jax.experimental.pallas import tpu as pltpu
from jax.experimental.pallas import tpu_sc as plsc

H = 16
MAXD = 5
EF = 3
NUM_SPATIAL = 512
VOCAB = 1536 * 8 + 1
VP = 12304
TROWS = 6 * VP
B, N = 32, 64
NPAIR = B * N * N

NW = 32
PAIRS_PER_W = NPAIR // NW
CHUNK = 256
NCHUNK = PAIRS_PER_W // CHUNK
ESLICE = CHUNK * 15 // 128
SSLICE = CHUNK // 128

_OFFPAT = np.array([((q % 15) // 3) * VP for q in range(240)] + [0] * 16,
                   dtype=np.int32)


def _spfac(sp):
    s = jnp.where(sp == 0, 1, sp)
    s = jnp.where(s > 1, s - 1, s)
    return jnp.clip(s, 0, MAXD)


def _build_table(edge_enc_w, edge_dis_w, spatial_enc_w):
    def body(e_ref, w_ref, s_ref, t_ref):
        E = e_ref[...]
        W5 = w_ref[...].reshape(-1, H, H)[:MAXD]
        mats = [jnp.dot(E, W5[d], preferred_element_type=jnp.float32) / 3.0
                for d in range(MAXD)]
        M = jnp.stack(mats, axis=0)
        M = jnp.pad(M, ((0, 0), (0, VP - VOCAB), (0, 0)))
        s_iota = lax.broadcasted_iota(jnp.int32, (NUM_SPATIAL,), 0)
        fac = _spfac(s_iota).astype(jnp.float32)
        S2 = s_ref[...] * fac[:, None]
        S2 = jnp.pad(S2, ((0, VP - NUM_SPATIAL), (0, 0)))[None]
        t_ref[...] = jnp.concatenate([M, S2], axis=0)

    return pl.pallas_call(
        body,
        out_shape=jax.ShapeDtypeStruct((6, VP, H), jnp.float32),
    )(edge_enc_w, edge_dis_w, spatial_enc_w)


def _sc_gather(t_flat, ein4, sp4, offpat):
    mesh = plsc.VectorSubcoreMesh(core_axis_name="c", subcore_axis_name="s")

    @functools.partial(
        pl.kernel,
        mesh=mesh,
        compiler_params=pltpu.CompilerParams(use_tc_tiling_on_sc=False),
        out_type=jax.ShapeDtypeStruct((NW, NCHUNK, CHUNK, H), jnp.float32),
        scratch_types=[
            pltpu.VMEM((ESLICE, 128), jnp.int32),
            pltpu.VMEM((SSLICE, 128), jnp.int32),
            pltpu.VMEM((CHUNK * 15, H), jnp.float32),
            pltpu.VMEM((CHUNK, H), jnp.float32),
            pltpu.VMEM((CHUNK, H), jnp.float32),
            pltpu.VMEM((256,), jnp.int32),
            pltpu.SemaphoreType.DMA,
        ],
    )
    def k(t_hbm, ein_hbm, sp_hbm, off_hbm, out_hbm,
          eidx, sidx, erows, srows, outbuf, offv, sem):
        w = lax.axis_index("s") * 2 + lax.axis_index("c")
        pltpu.sync_copy(off_hbm, offv)

        def chunk_body(c, _):
            pltpu.sync_copy(ein_hbm.at[w, c], eidx)
            pltpu.sync_copy(sp_hbm.at[w, c], sidx)

            def add_off(j, _):
                r = j // 8
                col = (j % 8) * 16
                v = eidx[r, pl.ds(col, 16)]
                o = offv[pl.ds((16 * j) % 240, 16)]
                eidx[r, pl.ds(col, 16)] = v + o
                return 0

            lax.fori_loop(0, ESLICE * 8, add_off, 0)

            def add_soff(j, _):
                r = j // 8
                col = (j % 8) * 16
                sidx[r, pl.ds(col, 16)] = sidx[r, pl.ds(col, 16)] + (5 * VP)
                return 0

            lax.fori_loop(0, SSLICE * 8, add_soff, 0)

            handles = []
            for s in range(ESLICE):
                handles.append(pltpu.async_copy(
                    t_hbm.at[eidx.at[s]], erows.at[pl.ds(s * 128, 128)], sem))
            for s in range(SSLICE):
                handles.append(pltpu.async_copy(
                    t_hbm.at[sidx.at[s]], srows.at[pl.ds(s * 128, 128)], sem))
            for h in handles:
                h.wait()

            def pair_body(p, _):
                base = p * 15
                acc = srows[p]
                for r in range(15):
                    acc = acc + erows[base + r]
                outbuf[p] = acc
                return 0

            lax.fori_loop(0, CHUNK, pair_body, 0)
            pltpu.sync_copy(outbuf, out_hbm.at[w, c])
            return 0

        lax.fori_loop(0, NCHUNK, chunk_body, 0)

    return k(t_flat, ein4, sp4, offpat)


def _assemble(esum, attn_bias, spatial_pos, vdist_w):
    def body(es_ref, ab_ref, sp_ref, vd_ref, out_ref):
        sp = sp_ref[0]
        scale = 1.0 / _spfac(sp).astype(jnp.float32)
        es = es_ref[0]
        interior = es * scale[:, :, None]
        intT = interior.reshape(N * N, H).T.reshape(H, N, N)
        ab = ab_ref[0]
        t = vd_ref[0]
        ii = lax.broadcasted_iota(jnp.int32, (H, N + 1, N + 1), 1)
        jj = lax.broadcasted_iota(jnp.int32, (H, N + 1, N + 1), 2)
        border = (ii == 0) | (jj == 0)
        int_p = jnp.pad(intT, ((0, 0), (1, 0), (1, 0)))
        out_ref[0] = 2.0 * ab[None] + jnp.where(border, t[:, None, None], int_p)

    return pl.pallas_call(
        body,
        grid=(B,),
        in_specs=[
            pl.BlockSpec((1, N, N, H), lambda b: (b, 0, 0, 0)),
            pl.BlockSpec((1, N + 1, N + 1), lambda b: (b, 0, 0)),
            pl.BlockSpec((1, N, N), lambda b: (b, 0, 0)),
            pl.BlockSpec((1, H), lambda b: (0, 0)),
        ],
        out_specs=pl.BlockSpec((1, H, N + 1, N + 1), lambda b: (b, 0, 0, 0)),
        out_shape=jax.ShapeDtypeStruct((B, H, N + 1, N + 1), jnp.float32),
    )(esum, attn_bias, spatial_pos, vdist_w)


def kernel(frag_feature, attn_bias, spatial_pos, edge_input, attn_edge_type,
           edge_enc_w, edge_dis_w, spatial_enc_w, vdist_w):
    del frag_feature, attn_edge_type
    T = _build_table(edge_enc_w, edge_dis_w, spatial_enc_w)
    t_flat = T.reshape(TROWS, H)
    ein4 = edge_input.astype(jnp.int32).reshape(NW, NCHUNK, ESLICE, 128)
    sp4 = spatial_pos.astype(jnp.int32).reshape(NW, NCHUNK, SSLICE, 128)
    offpat = jnp.asarray(_OFFPAT)
    esum = _sc_gather(t_flat, ein4, sp4, offpat)
    esum = esum.reshape(B, N, N, H)
    return _assemble(esum, attn_bias, spatial_pos, vdist_w)
